# baseline (speedup 1.0000x reference)
import functools

import jax
import jax.numpy as jnp
from jax import lax
from jax.experimental import pallas as pl
from jax.experimental.pallas import tpu as pltpu
from jax.experimental.pallas import tpu_sc as plsc

N = 10000
E = 320000
D = 128
DH = D // 2
NC = 2
NS = 16
NW = NC * NS
CH = 80
EPW = E // NW
ROWS_PW = EPW // CH
SRC_SHIFT = 14
NPT = N // NS


def _sc_agg_body(with_deg, x_hbm, pk_hbm, *rest):
    if with_deg:
        (acc_hbm, deg_hbm, pk_v, src_b, dst_b, rows_a, rows_b, deg_v,
         sem_a, sem_b, acc_sh) = rest
    else:
        (acc_hbm, pk_v, src_b, dst_b, rows_a, rows_b, sem_a, sem_b,
         acc_sh) = rest
        deg_hbm = deg_v = None
    c = lax.axis_index("c")
    s = lax.axis_index("s")
    g = c * NS + s

    z16 = jnp.zeros((16,), jnp.float32)
    ones16 = jnp.full((16,), 1.0, jnp.float32)

    if with_deg:
        @pl.loop(0, N, step=16)
        def _(i):
            deg_v[pl.ds(i, 16)] = z16

    @pl.loop(0, CH)
    def _(r):
        for k in range(D // 16):
            rows_a[r, pl.ds(k * 16, 16)] = z16

    for r in range(NPT // CH):
        pltpu.sync_copy(rows_a, acc_sh.at[pl.ds(s * NPT + r * CH, CH)])
    rem = NPT % CH
    pltpu.sync_copy(rows_a.at[pl.ds(0, rem)],
                    acc_sh.at[pl.ds(s * NPT + NPT - rem, rem)])
    plsc.subcore_barrier()

    pltpu.sync_copy(pk_hbm.at[g], pk_v)

    def unpack(j, r):
        for k in range(CH // 16):
            p = pk_v[j, pl.ds(k * 16, 16)]
            d = p & ((1 << SRC_SHIFT) - 1)
            src_b[r, pl.ds(k * 16, 16)] = p >> SRC_SHIFT
            dst_b[r, pl.ds(k * 16, 16)] = d
            if with_deg:
                plsc.addupdate_scatter(deg_v, [d], ones16)

    def gather(r, buf, sem):
        return pltpu.async_copy(x_hbm.at[src_b.at[r]], buf, sem)

    def wait_gather(r, buf, sem):
        pltpu.make_async_copy(x_hbm.at[src_b.at[r]], buf, sem).wait()

    def scatter(r, buf):
        pltpu.sync_copy(buf, acc_sh.at[dst_b.at[r]], add=True)

    unpack(0, 0)
    gather(0, rows_a, sem_a)

    @pl.loop(0, (ROWS_PW - 3) // 2)
    def _(t):
        j = 2 * t
        unpack(j + 1, 1)
        wait_gather(0, rows_a, sem_a)
        gather(1, rows_b, sem_b)
        scatter(0, rows_a)
        unpack(j + 2, 0)
        wait_gather(1, rows_b, sem_b)
        gather(0, rows_a, sem_a)
        scatter(1, rows_b)

    jf = ROWS_PW - 3
    unpack(jf + 1, 1)
    wait_gather(0, rows_a, sem_a)
    gather(1, rows_b, sem_b)
    scatter(0, rows_a)
    unpack(jf + 2, 0)
    wait_gather(1, rows_b, sem_b)
    gather(0, rows_a, sem_a)
    scatter(1, rows_b)
    wait_gather(0, rows_a, sem_a)
    scatter(0, rows_a)

    plsc.subcore_barrier()
    pltpu.sync_copy(acc_sh.at[pl.ds(s * NPT, NPT)], acc_hbm.at[c, s])
    if with_deg:
        pltpu.sync_copy(deg_v, deg_hbm.at[pl.ds(g * N, N)])


def _sc_deg_body(dst_hbm, deg_hbm, dst_v, deg_v):
    c = lax.axis_index("c")
    s = lax.axis_index("s")
    g = c * NS + s

    z16 = jnp.zeros((16,), jnp.float32)

    @pl.loop(0, N, step=16)
    def _(i):
        deg_v[pl.ds(i, 16)] = z16

    pltpu.sync_copy(dst_hbm.at[g], dst_v)

    ones16 = jnp.full((16,), 1.0, jnp.float32)

    @pl.loop(0, EPW // 16)
    def _(j):
        plsc.addupdate_scatter(deg_v, [dst_v[j]], ones16)

    pltpu.sync_copy(deg_v, deg_hbm.at[pl.ds(g * N, N)])


@functools.cache
def _sc_kernels():
    mesh = plsc.VectorSubcoreMesh(
        core_axis_name="c", subcore_axis_name="s",
        num_cores=NC, num_subcores=NS)
    params = pltpu.CompilerParams(needs_layout_passes=False,
                                  skip_device_barrier=True)
    common = [
        pltpu.VMEM((ROWS_PW, CH), jnp.int32),
        pltpu.VMEM((8, CH), jnp.int32),
        pltpu.VMEM((8, CH), jnp.int32),
        pltpu.VMEM((CH, D), jnp.float32),
        pltpu.VMEM((CH, D), jnp.float32),
    ]
    tail = [
        pltpu.SemaphoreType.DMA,
        pltpu.SemaphoreType.DMA,
        pltpu.VMEM_SHARED((N, D), jnp.float32),
    ]
    agg_deg = pl.kernel(
        functools.partial(_sc_agg_body, True),
        compiler_params=params,
        out_type=(jax.ShapeDtypeStruct((NC, NS, NPT, D), jnp.float32),
                  jax.ShapeDtypeStruct((NW * N,), jnp.float32)),
        mesh=mesh,
        scratch_types=common + [pltpu.VMEM((N,), jnp.float32)] + tail,
    )
    agg = pl.kernel(
        functools.partial(_sc_agg_body, False),
        compiler_params=params,
        out_type=jax.ShapeDtypeStruct((NC, NS, NPT, D), jnp.float32),
        mesh=mesh,
        scratch_types=common + tail,
    )
    return agg_deg, agg


BM = 400
_GRID = N // BM


def _tc_layer_body(relu_out, acc_ref, deg_ref, x_ref, wl_ref, bl_ref, wr_ref,
                   *outs):
    deg = jnp.sum(deg_ref[...], axis=0)
    invd = 1.0 / jnp.maximum(deg, 1.0)
    mean = (acc_ref[0] + acc_ref[1]) * invd
    h1 = (lax.dot_general(mean, wl_ref[...], (((1,), (1,)), ((), ())),
                          preferred_element_type=jnp.float32)
          + bl_ref[...]
          + lax.dot_general(x_ref[...], wr_ref[...], (((1,), (1,)), ((), ())),
                            preferred_element_type=jnp.float32))
    outs[0][...] = h1
    if relu_out:
        outs[1][...] = jnp.maximum(h1, 0.0)


def _make_tc(relu_out):
    n_out = 2 if relu_out else 1
    return pl.pallas_call(
        functools.partial(_tc_layer_body, relu_out),
        grid=(_GRID,),
        in_specs=[
            pl.BlockSpec((NC, BM, D), lambda i: (0, i, 0)),
            pl.BlockSpec((NW, BM, 1), lambda i: (0, i, 0)),
            pl.BlockSpec((BM, D), lambda i: (i, 0)),
            pl.BlockSpec((D, D), lambda i: (0, 0)),
            pl.BlockSpec((1, D), lambda i: (0, 0)),
            pl.BlockSpec((D, D), lambda i: (0, 0)),
        ],
        out_specs=[pl.BlockSpec((BM, D), lambda i: (i, 0))] * n_out,
        out_shape=[jax.ShapeDtypeStruct((N, D), jnp.float32)] * n_out,
    )


_tc_layer_relu = _make_tc(True)
_tc_layer_last = _make_tc(False)


def kernel(x, edge_index, W_l0, b_l0, W_r0, W_l1, b_l1, W_r1):
    sc_agg_deg, sc_agg = _sc_kernels()
    srcf = edge_index[0].astype(jnp.int32)
    dstf = edge_index[1].astype(jnp.int32)
    pk = ((srcf << SRC_SHIFT) | dstf).reshape(NW, ROWS_PW, CH)
    acc0, degp = sc_agg_deg(x, pk)
    acc0 = acc0.reshape(NC, N, D)
    deg = degp.reshape(NW, N, 1)
    h1, h = _tc_layer_relu(acc0, deg, x, W_l0, b_l0.reshape(1, D), W_r0)
    acc1 = sc_agg(h, pk).reshape(NC, N, D)
    (h2,) = _tc_layer_last(acc1, deg, h, W_l1, b_l1.reshape(1, D), W_r1)
    return (h1, h2)


# speedup vs baseline: 1.1013x; 1.1013x over previous
import functools

import jax
import jax.numpy as jnp
from jax import lax
from jax.experimental import pallas as pl
from jax.experimental.pallas import tpu as pltpu
from jax.experimental.pallas import tpu_sc as plsc

N = 10000
E = 320000
D = 128
DH = D // 2
NC = 2
NS = 16
NW = NC * NS
CH = 80
EPW = E // NW
ROWS_PW = EPW // CH
SRC_SHIFT = 14
NPT = N // NS


def _sc_agg_body(with_deg, x_hbm, pk_hbm, *rest):
    if with_deg:
        (acc_hbm, deg_hbm, pk_v, src_b, dst_b, rows_a, rows_b, deg_v,
         sem_a, sem_b, acc_sh) = rest
    else:
        (acc_hbm, pk_v, src_b, dst_b, rows_a, rows_b, sem_a, sem_b,
         acc_sh) = rest
        deg_hbm = deg_v = None
    c = lax.axis_index("c")
    s = lax.axis_index("s")
    g = c * NS + s

    z16 = jnp.zeros((16,), jnp.float32)
    ones16 = jnp.full((16,), 1.0, jnp.float32)

    if with_deg:
        @pl.loop(0, N, step=16)
        def _(i):
            deg_v[pl.ds(i, 16)] = z16

    @pl.loop(0, CH)
    def _(r):
        for k in range(D // 16):
            rows_a[r, pl.ds(k * 16, 16)] = z16

    for r in range(NPT // CH):
        pltpu.sync_copy(rows_a, acc_sh.at[pl.ds(s * NPT + r * CH, CH)])
    rem = NPT % CH
    pltpu.sync_copy(rows_a.at[pl.ds(0, rem)],
                    acc_sh.at[pl.ds(s * NPT + NPT - rem, rem)])
    plsc.subcore_barrier()

    pltpu.sync_copy(pk_hbm.at[g], pk_v)

    def unpack(j, r):
        for k in range(CH // 16):
            p = pk_v[j, pl.ds(k * 16, 16)]
            d = p & ((1 << SRC_SHIFT) - 1)
            src_b[r, pl.ds(k * 16, 16)] = p >> SRC_SHIFT
            dst_b[r, pl.ds(k * 16, 16)] = d
            if with_deg:
                plsc.addupdate_scatter(deg_v, [d], ones16)

    def gather(r, buf, sem):
        return pltpu.async_copy(x_hbm.at[src_b.at[r]], buf, sem)

    def wait_gather(r, buf, sem):
        pltpu.make_async_copy(x_hbm.at[src_b.at[r]], buf, sem).wait()

    def scatter(r, buf):
        pltpu.sync_copy(buf, acc_sh.at[dst_b.at[r]], add=True)

    unpack(0, 0)
    gather(0, rows_a, sem_a)

    @pl.loop(0, (ROWS_PW - 3) // 2)
    def _(t):
        j = 2 * t
        unpack(j + 1, 1)
        wait_gather(0, rows_a, sem_a)
        gather(1, rows_b, sem_b)
        scatter(0, rows_a)
        unpack(j + 2, 0)
        wait_gather(1, rows_b, sem_b)
        gather(0, rows_a, sem_a)
        scatter(1, rows_b)

    jf = ROWS_PW - 3
    unpack(jf + 1, 1)
    wait_gather(0, rows_a, sem_a)
    gather(1, rows_b, sem_b)
    scatter(0, rows_a)
    unpack(jf + 2, 0)
    wait_gather(1, rows_b, sem_b)
    gather(0, rows_a, sem_a)
    scatter(1, rows_b)
    wait_gather(0, rows_a, sem_a)
    scatter(0, rows_a)

    plsc.subcore_barrier()
    pltpu.sync_copy(acc_sh.at[pl.ds(s * NPT, NPT)], acc_hbm.at[c, s])
    if with_deg:
        pltpu.sync_copy(deg_v, deg_hbm.at[pl.ds(g * N, N)])


def _sc_deg_body(dst_hbm, deg_hbm, dst_v, deg_v):
    c = lax.axis_index("c")
    s = lax.axis_index("s")
    g = c * NS + s

    z16 = jnp.zeros((16,), jnp.float32)

    @pl.loop(0, N, step=16)
    def _(i):
        deg_v[pl.ds(i, 16)] = z16

    pltpu.sync_copy(dst_hbm.at[g], dst_v)

    ones16 = jnp.full((16,), 1.0, jnp.float32)

    @pl.loop(0, EPW // 16)
    def _(j):
        plsc.addupdate_scatter(deg_v, [dst_v[j]], ones16)

    pltpu.sync_copy(deg_v, deg_hbm.at[pl.ds(g * N, N)])


@functools.cache
def _sc_kernels():
    mesh = plsc.VectorSubcoreMesh(
        core_axis_name="c", subcore_axis_name="s",
        num_cores=NC, num_subcores=NS)
    params = pltpu.CompilerParams(needs_layout_passes=False,
                                  skip_device_barrier=True)
    common = [
        pltpu.VMEM((ROWS_PW, CH), jnp.int32),
        pltpu.VMEM((8, CH), jnp.int32),
        pltpu.VMEM((8, CH), jnp.int32),
        pltpu.VMEM((CH, D), jnp.float32),
        pltpu.VMEM((CH, D), jnp.float32),
    ]
    tail = [
        pltpu.SemaphoreType.DMA,
        pltpu.SemaphoreType.DMA,
        pltpu.VMEM_SHARED((N, D), jnp.float32),
    ]
    deg = pl.kernel(
        _sc_deg_body,
        compiler_params=params,
        out_type=jax.ShapeDtypeStruct((NW * N,), jnp.float32),
        mesh=mesh,
        scratch_types=[
            pltpu.VMEM((EPW // 16, 16), jnp.int32),
            pltpu.VMEM((N,), jnp.float32),
        ],
    )
    agg = pl.kernel(
        functools.partial(_sc_agg_body, False),
        compiler_params=params,
        out_type=jax.ShapeDtypeStruct((NC, NS, NPT, D), jnp.float32),
        mesh=mesh,
        scratch_types=common + tail,
    )
    return agg, deg


BM = 400
_GRID = N // BM


def _tc_layer_body(relu_out, acc_ref, deg_ref, x_ref, wl_ref, bl_ref, wr_ref,
                   *outs):
    deg = jnp.sum(deg_ref[...], axis=0)
    invd = 1.0 / jnp.maximum(deg, 1.0)
    mean = (acc_ref[0] + acc_ref[1]) * invd
    h1 = (lax.dot_general(mean, wl_ref[...], (((1,), (1,)), ((), ())),
                          preferred_element_type=jnp.float32)
          + bl_ref[...]
          + lax.dot_general(x_ref[...], wr_ref[...], (((1,), (1,)), ((), ())),
                            preferred_element_type=jnp.float32))
    outs[0][...] = h1
    if relu_out:
        outs[1][...] = jnp.maximum(h1, 0.0)


def _make_tc(relu_out):
    n_out = 2 if relu_out else 1
    return pl.pallas_call(
        functools.partial(_tc_layer_body, relu_out),
        grid=(_GRID,),
        in_specs=[
            pl.BlockSpec((NC, BM, D), lambda i: (0, i, 0)),
            pl.BlockSpec((NW, BM, 1), lambda i: (0, i, 0)),
            pl.BlockSpec((BM, D), lambda i: (i, 0)),
            pl.BlockSpec((D, D), lambda i: (0, 0)),
            pl.BlockSpec((1, D), lambda i: (0, 0)),
            pl.BlockSpec((D, D), lambda i: (0, 0)),
        ],
        out_specs=[pl.BlockSpec((BM, D), lambda i: (i, 0))] * n_out,
        out_shape=[jax.ShapeDtypeStruct((N, D), jnp.float32)] * n_out,
    )


_tc_layer_relu = _make_tc(True)
_tc_layer_last = _make_tc(False)


def kernel(x, edge_index, W_l0, b_l0, W_r0, W_l1, b_l1, W_r1):
    sc_agg, sc_deg = _sc_kernels()
    srcf = edge_index[0].astype(jnp.int32)
    dstf = edge_index[1].astype(jnp.int32)
    pk = ((srcf << SRC_SHIFT) | dstf).reshape(NW, ROWS_PW, CH)
    degp = sc_deg(dstf.reshape(NW, EPW // 16, 16))
    deg = degp.reshape(NW, N, 1)
    acc0 = sc_agg(x, pk).reshape(NC, N, D)
    h1, h = _tc_layer_relu(acc0, deg, x, W_l0, b_l0.reshape(1, D), W_r0)
    acc1 = sc_agg(h, pk).reshape(NC, N, D)
    (h2,) = _tc_layer_last(acc1, deg, h, W_l1, b_l1.reshape(1, D), W_r1)
    return (h1, h2)


# speedup vs baseline: 1.4080x; 1.2785x over previous
import functools

import jax
import jax.numpy as jnp
from jax import lax
from jax.experimental import pallas as pl
from jax.experimental.pallas import tpu as pltpu
from jax.experimental.pallas import tpu_sc as plsc

N = 10000
E = 320000
D = 128
DH = D // 2
NC = 2
NS = 16
NW = NC * NS
CH = 80
EPW = E // NW
ROWS_PW = EPW // CH
SRC_SHIFT = 14
NPT = N // NS


def _sc_agg_body(with_deg, x_hbm, pk_hbm, *rest):
    if with_deg:
        (acc_hbm, deg_hbm, pk_v, src_b, dst_b, rows_a, rows_b, deg_v,
         sem_a, sem_b, acc_sh) = rest
    else:
        (acc_hbm, pk_v, src_b, dst_b, rows_a, rows_b, sem_a, sem_b,
         acc_sh) = rest
        deg_hbm = deg_v = None
    c = lax.axis_index("c")
    s = lax.axis_index("s")
    g = c * NS + s

    z16 = jnp.zeros((16,), jnp.float32)
    ones16 = jnp.full((16,), 1.0, jnp.float32)

    if with_deg:
        @pl.loop(0, N, step=16)
        def _(i):
            deg_v[pl.ds(i, 16)] = z16

    @pl.loop(0, CH)
    def _(r):
        for k in range(D // 16):
            rows_a[r, pl.ds(k * 16, 16)] = z16

    for r in range(NPT // CH):
        pltpu.sync_copy(rows_a, acc_sh.at[pl.ds(s * NPT + r * CH, CH)])
    rem = NPT % CH
    pltpu.sync_copy(rows_a.at[pl.ds(0, rem)],
                    acc_sh.at[pl.ds(s * NPT + NPT - rem, rem)])
    plsc.subcore_barrier()

    pltpu.sync_copy(pk_hbm.at[g], pk_v)

    def unpack(j, r):
        for k in range(CH // 16):
            p = pk_v[j, pl.ds(k * 16, 16)]
            d = p & ((1 << SRC_SHIFT) - 1)
            src_b[r, pl.ds(k * 16, 16)] = p >> SRC_SHIFT
            dst_b[r, pl.ds(k * 16, 16)] = d
            if with_deg:
                plsc.addupdate_scatter(deg_v, [d], ones16)

    def gather(r, buf, sem):
        return pltpu.async_copy(x_hbm.at[src_b.at[r]], buf, sem)

    def wait_gather(r, buf, sem):
        pltpu.make_async_copy(x_hbm.at[src_b.at[r]], buf, sem).wait()

    def scatter(r, buf):
        pltpu.sync_copy(buf, acc_sh.at[dst_b.at[r]], add=True)

    unpack(0, 0)
    gather(0, rows_a, sem_a)

    @pl.loop(0, (ROWS_PW - 3) // 2)
    def _(t):
        j = 2 * t
        unpack(j + 1, 1)
        wait_gather(0, rows_a, sem_a)
        gather(1, rows_b, sem_b)
        scatter(0, rows_a)
        unpack(j + 2, 0)
        wait_gather(1, rows_b, sem_b)
        gather(0, rows_a, sem_a)
        scatter(1, rows_b)

    jf = ROWS_PW - 3
    unpack(jf + 1, 1)
    wait_gather(0, rows_a, sem_a)
    gather(1, rows_b, sem_b)
    scatter(0, rows_a)
    unpack(jf + 2, 0)
    wait_gather(1, rows_b, sem_b)
    gather(0, rows_a, sem_a)
    scatter(1, rows_b)
    wait_gather(0, rows_a, sem_a)
    scatter(0, rows_a)

    plsc.subcore_barrier()
    a8 = pl.multiple_of((s * NPT + 7) // 8 * 8, 8)
    b8 = ((s + 1) * NPT + 7) // 8 * 8
    pltpu.sync_copy(acc_sh.at[pl.ds(a8, 624)], acc_hbm.at[c, pl.ds(a8, 624)])

    @pl.when(b8 - a8 > 624)
    def _():
        t8 = pl.multiple_of(a8 + 624, 8)
        pltpu.sync_copy(acc_sh.at[pl.ds(t8, 8)], acc_hbm.at[c, pl.ds(t8, 8)])
    if with_deg:
        pltpu.sync_copy(deg_v, deg_hbm.at[pl.ds(g * N, N)])


def _sc_deg_body(dst_hbm, deg_hbm, dst_v, deg_v):
    c = lax.axis_index("c")
    s = lax.axis_index("s")
    g = c * NS + s

    z16 = jnp.zeros((16,), jnp.float32)

    @pl.loop(0, N, step=16)
    def _(i):
        deg_v[pl.ds(i, 16)] = z16

    pltpu.sync_copy(dst_hbm.at[g], dst_v)

    ones16 = jnp.full((16,), 1.0, jnp.float32)

    @pl.loop(0, EPW // 16)
    def _(j):
        plsc.addupdate_scatter(deg_v, [dst_v[j]], ones16)

    pltpu.sync_copy(deg_v, deg_hbm.at[pl.ds(g * N, N)])


@functools.cache
def _sc_kernels():
    mesh = plsc.VectorSubcoreMesh(
        core_axis_name="c", subcore_axis_name="s",
        num_cores=NC, num_subcores=NS)
    params = pltpu.CompilerParams(needs_layout_passes=False,
                                  skip_device_barrier=True)
    common = [
        pltpu.VMEM((ROWS_PW, CH), jnp.int32),
        pltpu.VMEM((8, CH), jnp.int32),
        pltpu.VMEM((8, CH), jnp.int32),
        pltpu.VMEM((CH, D), jnp.float32),
        pltpu.VMEM((CH, D), jnp.float32),
    ]
    tail = [
        pltpu.SemaphoreType.DMA,
        pltpu.SemaphoreType.DMA,
        pltpu.VMEM_SHARED((N, D), jnp.float32),
    ]
    deg = pl.kernel(
        _sc_deg_body,
        compiler_params=params,
        out_type=jax.ShapeDtypeStruct((NW * N,), jnp.float32),
        mesh=mesh,
        scratch_types=[
            pltpu.VMEM((EPW // 16, 16), jnp.int32),
            pltpu.VMEM((N,), jnp.float32),
        ],
    )
    agg = pl.kernel(
        functools.partial(_sc_agg_body, False),
        compiler_params=params,
        out_type=jax.ShapeDtypeStruct((NC, N, D), jnp.float32),
        mesh=mesh,
        scratch_types=common + tail,
    )
    return agg, deg


BM = 400
_GRID = N // BM


def _tc_layer_body(relu_out, acc_ref, deg_ref, x_ref, wl_ref, bl_ref, wr_ref,
                   *outs):
    deg = jnp.sum(deg_ref[...], axis=0)
    invd = 1.0 / jnp.maximum(deg, 1.0)
    mean = (acc_ref[0] + acc_ref[1]) * invd
    h1 = (lax.dot_general(mean, wl_ref[...], (((1,), (1,)), ((), ())),
                          preferred_element_type=jnp.float32)
          + bl_ref[...]
          + lax.dot_general(x_ref[...], wr_ref[...], (((1,), (1,)), ((), ())),
                            preferred_element_type=jnp.float32))
    outs[0][...] = h1
    if relu_out:
        outs[1][...] = jnp.maximum(h1, 0.0)


def _make_tc(relu_out):
    n_out = 2 if relu_out else 1
    return pl.pallas_call(
        functools.partial(_tc_layer_body, relu_out),
        grid=(_GRID,),
        in_specs=[
            pl.BlockSpec((NC, BM, D), lambda i: (0, i, 0)),
            pl.BlockSpec((NW, BM, 1), lambda i: (0, i, 0)),
            pl.BlockSpec((BM, D), lambda i: (i, 0)),
            pl.BlockSpec((D, D), lambda i: (0, 0)),
            pl.BlockSpec((1, D), lambda i: (0, 0)),
            pl.BlockSpec((D, D), lambda i: (0, 0)),
        ],
        out_specs=[pl.BlockSpec((BM, D), lambda i: (i, 0))] * n_out,
        out_shape=[jax.ShapeDtypeStruct((N, D), jnp.float32)] * n_out,
    )


_tc_layer_relu = _make_tc(True)
_tc_layer_last = _make_tc(False)


def kernel(x, edge_index, W_l0, b_l0, W_r0, W_l1, b_l1, W_r1):
    sc_agg, sc_deg = _sc_kernels()
    srcf = edge_index[0].astype(jnp.int32)
    dstf = edge_index[1].astype(jnp.int32)
    pk = ((srcf << SRC_SHIFT) | dstf).reshape(NW, ROWS_PW, CH)
    degp = sc_deg(dstf.reshape(NW, EPW // 16, 16))
    deg = degp.reshape(NW, N, 1)
    acc0 = sc_agg(x, pk)
    h1, h = _tc_layer_relu(acc0, deg, x, W_l0, b_l0.reshape(1, D), W_r0)
    acc1 = sc_agg(h, pk)
    (h2,) = _tc_layer_last(acc1, deg, h, W_l1, b_l1.reshape(1, D), W_r1)
    return (h1, h2)


# speedup vs baseline: 1.7362x; 1.2331x over previous
import functools

import jax
import jax.numpy as jnp
from jax import lax
from jax.experimental import pallas as pl
from jax.experimental.pallas import tpu as pltpu
from jax.experimental.pallas import tpu_sc as plsc

N = 10000
E = 320000
D = 128
DH = D // 2
NC = 2
NS = 16
NW = NC * NS
CH = 80
EPW = E // NW
ROWS_PW = EPW // CH
SRC_SHIFT = 14
NPT = N // NS


def _sc_agg_body(with_deg, x_hbm, pk_hbm, *rest):
    if with_deg:
        (acc_hbm, deg_hbm, pk_v, src_b, dst_b, rows_a, rows_b, deg_v,
         sem_a, sem_b, acc_sh) = rest
    else:
        (acc_hbm, pk_v, src_b, dst_b, rows_a, rows_b, sem_a, sem_b,
         acc_sh) = rest
        deg_hbm = deg_v = None
    c = lax.axis_index("c")
    s = lax.axis_index("s")
    g = c * NS + s

    z16 = jnp.zeros((16,), jnp.float32)
    ones16 = jnp.full((16,), 1.0, jnp.float32)

    if with_deg:
        @pl.loop(0, N, step=16)
        def _(i):
            deg_v[pl.ds(i, 16)] = z16

    @pl.loop(0, CH)
    def _(r):
        for k in range(D // 16):
            rows_a[r, pl.ds(k * 16, 16)] = z16

    for r in range(NPT // CH):
        pltpu.sync_copy(rows_a, acc_sh.at[pl.ds(s * NPT + r * CH, CH)])
    rem = NPT % CH
    pltpu.sync_copy(rows_a.at[pl.ds(0, rem)],
                    acc_sh.at[pl.ds(s * NPT + NPT - rem, rem)])
    plsc.subcore_barrier()

    pltpu.sync_copy(pk_hbm.at[g], pk_v)

    def unpack(j, r):
        for k in range(CH // 16):
            p = pk_v[j, pl.ds(k * 16, 16)]
            d = p & ((1 << SRC_SHIFT) - 1)
            src_b[r, pl.ds(k * 16, 16)] = p >> SRC_SHIFT
            dst_b[r, pl.ds(k * 16, 16)] = d
            if with_deg:
                plsc.addupdate_scatter(deg_v, [d], ones16)

    def gather(r, buf, sem):
        return pltpu.async_copy(x_hbm.at[src_b.at[r]], buf, sem)

    def wait_gather(r, buf, sem):
        pltpu.make_async_copy(x_hbm.at[src_b.at[r]], buf, sem).wait()

    def scatter(r, buf):
        pltpu.sync_copy(buf, acc_sh.at[dst_b.at[r]], add=True)

    unpack(0, 0)
    gather(0, rows_a, sem_a)

    @pl.loop(0, (ROWS_PW - 3) // 2)
    def _(t):
        j = 2 * t
        unpack(j + 1, 1)
        wait_gather(0, rows_a, sem_a)
        gather(1, rows_b, sem_b)
        scatter(0, rows_a)
        unpack(j + 2, 0)
        wait_gather(1, rows_b, sem_b)
        gather(0, rows_a, sem_a)
        scatter(1, rows_b)

    jf = ROWS_PW - 3
    unpack(jf + 1, 1)
    wait_gather(0, rows_a, sem_a)
    gather(1, rows_b, sem_b)
    scatter(0, rows_a)
    unpack(jf + 2, 0)
    wait_gather(1, rows_b, sem_b)
    gather(0, rows_a, sem_a)
    scatter(1, rows_b)
    wait_gather(0, rows_a, sem_a)
    scatter(0, rows_a)

    plsc.subcore_barrier()
    a8 = pl.multiple_of((s * NPT + 7) // 8 * 8, 8)
    b8 = ((s + 1) * NPT + 7) // 8 * 8
    pltpu.sync_copy(acc_sh.at[pl.ds(a8, 624)], acc_hbm.at[c, pl.ds(a8, 624)])

    @pl.when(b8 - a8 > 624)
    def _():
        t8 = pl.multiple_of(a8 + 624, 8)
        pltpu.sync_copy(acc_sh.at[pl.ds(t8, 8)], acc_hbm.at[c, pl.ds(t8, 8)])
    if with_deg:
        pltpu.sync_copy(deg_v, deg_hbm.at[pl.ds(g * N, N)])


def _sc_deg_body(dst_hbm, deg_hbm, dst_v, deg_v):
    c = lax.axis_index("c")
    s = lax.axis_index("s")
    g = c * NS + s

    z16 = jnp.zeros((16,), jnp.float32)

    @pl.loop(0, N, step=16)
    def _(i):
        deg_v[pl.ds(i, 16)] = z16

    pltpu.sync_copy(dst_hbm.at[g], dst_v)

    ones16 = jnp.full((16,), 1.0, jnp.float32)

    @pl.loop(0, EPW // 16)
    def _(j):
        plsc.addupdate_scatter(deg_v, [dst_v[j]], ones16)

    pltpu.sync_copy(deg_v, deg_hbm.at[pl.ds(g * N, N)])


@functools.cache
def _sc_kernels():
    mesh = plsc.VectorSubcoreMesh(
        core_axis_name="c", subcore_axis_name="s",
        num_cores=NC, num_subcores=NS)
    params = pltpu.CompilerParams(needs_layout_passes=False,
                                  skip_device_barrier=True)
    common = [
        pltpu.VMEM((ROWS_PW, CH), jnp.int32),
        pltpu.VMEM((8, CH), jnp.int32),
        pltpu.VMEM((8, CH), jnp.int32),
        pltpu.VMEM((CH, D), jnp.float32),
        pltpu.VMEM((CH, D), jnp.float32),
    ]
    tail = [
        pltpu.SemaphoreType.DMA,
        pltpu.SemaphoreType.DMA,
        pltpu.VMEM_SHARED((N, D), jnp.float32),
    ]
    deg = pl.kernel(
        _sc_deg_body,
        compiler_params=params,
        out_type=jax.ShapeDtypeStruct((NW * N,), jnp.float32),
        mesh=mesh,
        scratch_types=[
            pltpu.VMEM((EPW // 16, 16), jnp.int32),
            pltpu.VMEM((N,), jnp.float32),
        ],
    )
    agg = pl.kernel(
        functools.partial(_sc_agg_body, False),
        compiler_params=params,
        out_type=jax.ShapeDtypeStruct((NC, N, D), jnp.float32),
        mesh=mesh,
        scratch_types=common + tail,
    )
    return agg, deg


BM = 400
_GRID = N // BM


def _tc_layer_body(relu_out, acc_ref, deg_ref, x_ref, wl_ref, bl_ref, wr_ref,
                   *outs):
    deg = jnp.sum(deg_ref[...], axis=1, keepdims=True)
    invd = 1.0 / jnp.maximum(deg, 1.0)
    mean = (acc_ref[0] + acc_ref[1]) * invd
    h1 = (lax.dot_general(mean, wl_ref[...], (((1,), (1,)), ((), ())),
                          preferred_element_type=jnp.float32)
          + bl_ref[...]
          + lax.dot_general(x_ref[...], wr_ref[...], (((1,), (1,)), ((), ())),
                            preferred_element_type=jnp.float32))
    outs[0][...] = h1
    if relu_out:
        outs[1][...] = jnp.maximum(h1, 0.0)


def _make_tc(relu_out):
    n_out = 2 if relu_out else 1
    return pl.pallas_call(
        functools.partial(_tc_layer_body, relu_out),
        grid=(_GRID,),
        in_specs=[
            pl.BlockSpec((NC, BM, D), lambda i: (0, i, 0)),
            pl.BlockSpec((BM, NW), lambda i: (i, 0)),
            pl.BlockSpec((BM, D), lambda i: (i, 0)),
            pl.BlockSpec((D, D), lambda i: (0, 0)),
            pl.BlockSpec((1, D), lambda i: (0, 0)),
            pl.BlockSpec((D, D), lambda i: (0, 0)),
        ],
        out_specs=[pl.BlockSpec((BM, D), lambda i: (i, 0))] * n_out,
        out_shape=[jax.ShapeDtypeStruct((N, D), jnp.float32)] * n_out,
    )


_tc_layer_relu = _make_tc(True)
_tc_layer_last = _make_tc(False)


def kernel(x, edge_index, W_l0, b_l0, W_r0, W_l1, b_l1, W_r1):
    sc_agg, sc_deg = _sc_kernels()
    srcf = edge_index[0].astype(jnp.int32)
    dstf = edge_index[1].astype(jnp.int32)
    pk = ((srcf << SRC_SHIFT) | dstf).reshape(NW, ROWS_PW, CH)
    degp = sc_deg(dstf.reshape(NW, EPW // 16, 16))
    deg = degp.reshape(NW, N).T
    acc0 = sc_agg(x, pk)
    h1, h = _tc_layer_relu(acc0, deg, x, W_l0, b_l0.reshape(1, D), W_r0)
    acc1 = sc_agg(h, pk)
    (h2,) = _tc_layer_last(acc1, deg, h, W_l1, b_l1.reshape(1, D), W_r1)
    return (h1, h2)


# speedup vs baseline: 1.7801x; 1.0253x over previous
import functools

import jax
import jax.numpy as jnp
from jax import lax
from jax.experimental import pallas as pl
from jax.experimental.pallas import tpu as pltpu
from jax.experimental.pallas import tpu_sc as plsc

N = 10000
E = 320000
D = 128
DH = D // 2
NC = 2
NS = 16
NW = NC * NS
CH = 80
EPW = E // NW
ROWS_PW = EPW // CH
SRC_SHIFT = 14
NPT = N // NS


def _sc_agg_body(with_deg, x_hbm, pk_hbm, *rest):
    if with_deg:
        (acc_hbm, deg_hbm, pk_v, src_b, dst_b, rows_a, rows_b, deg_v,
         sem_a, sem_b, acc_sh) = rest
    else:
        (acc_hbm, pk_v, src_b, dst_b, rows_a, rows_b, sem_a, sem_b,
         acc_sh) = rest
        deg_hbm = deg_v = None
    c = lax.axis_index("c")
    s = lax.axis_index("s")
    g = c * NS + s

    z16 = jnp.zeros((16,), jnp.float32)
    ones16 = jnp.full((16,), 1.0, jnp.float32)

    if with_deg:
        @pl.loop(0, N, step=16)
        def _(i):
            deg_v[pl.ds(i, 16)] = z16

    @pl.loop(0, CH)
    def _(r):
        for k in range(D // 16):
            rows_a[r, pl.ds(k * 16, 16)] = z16

    for r in range(NPT // CH):
        pltpu.sync_copy(rows_a, acc_sh.at[pl.ds(s * NPT + r * CH, CH)])
    rem = NPT % CH
    pltpu.sync_copy(rows_a.at[pl.ds(0, rem)],
                    acc_sh.at[pl.ds(s * NPT + NPT - rem, rem)])
    plsc.subcore_barrier()

    pltpu.sync_copy(pk_hbm.at[pl.ds(g * EPW, EPW)], pk_v)

    def unpack(j, r):
        for k in range(CH // 16):
            p = pk_v[pl.ds(j * CH + k * 16, 16)]
            d = p & ((1 << SRC_SHIFT) - 1)
            src_b[r, pl.ds(k * 16, 16)] = p >> SRC_SHIFT
            dst_b[r, pl.ds(k * 16, 16)] = d
            if with_deg:
                plsc.addupdate_scatter(deg_v, [d], ones16)

    def gather(r, buf, sem):
        return pltpu.async_copy(x_hbm.at[src_b.at[r]], buf, sem)

    def wait_gather(r, buf, sem):
        pltpu.make_async_copy(x_hbm.at[src_b.at[r]], buf, sem).wait()

    def scatter(r, buf):
        pltpu.sync_copy(buf, acc_sh.at[dst_b.at[r]], add=True)

    unpack(0, 0)
    gather(0, rows_a, sem_a)

    @pl.loop(0, (ROWS_PW - 3) // 2)
    def _(t):
        j = 2 * t
        unpack(j + 1, 1)
        wait_gather(0, rows_a, sem_a)
        gather(1, rows_b, sem_b)
        scatter(0, rows_a)
        unpack(j + 2, 0)
        wait_gather(1, rows_b, sem_b)
        gather(0, rows_a, sem_a)
        scatter(1, rows_b)

    jf = ROWS_PW - 3
    unpack(jf + 1, 1)
    wait_gather(0, rows_a, sem_a)
    gather(1, rows_b, sem_b)
    scatter(0, rows_a)
    unpack(jf + 2, 0)
    wait_gather(1, rows_b, sem_b)
    gather(0, rows_a, sem_a)
    scatter(1, rows_b)
    wait_gather(0, rows_a, sem_a)
    scatter(0, rows_a)

    plsc.subcore_barrier()
    a8 = pl.multiple_of((s * NPT + 7) // 8 * 8, 8)
    b8 = ((s + 1) * NPT + 7) // 8 * 8
    pltpu.sync_copy(acc_sh.at[pl.ds(a8, 624)], acc_hbm.at[c, pl.ds(a8, 624)])

    @pl.when(b8 - a8 > 624)
    def _():
        t8 = pl.multiple_of(a8 + 624, 8)
        pltpu.sync_copy(acc_sh.at[pl.ds(t8, 8)], acc_hbm.at[c, pl.ds(t8, 8)])
    if with_deg:
        pltpu.sync_copy(deg_v, deg_hbm.at[pl.ds(g * N, N)])


def _sc_deg_body(src_hbm, dst_hbm, deg_hbm, pk_hbm, src_v, dst_v, pk_v,
                 deg_v):
    c = lax.axis_index("c")
    s = lax.axis_index("s")
    g = c * NS + s

    z16 = jnp.zeros((16,), jnp.float32)

    @pl.loop(0, N, step=16)
    def _(i):
        deg_v[pl.ds(i, 16)] = z16

    pltpu.sync_copy(src_hbm.at[pl.ds(g * EPW, EPW)], src_v)
    pltpu.sync_copy(dst_hbm.at[pl.ds(g * EPW, EPW)], dst_v)

    ones16 = jnp.full((16,), 1.0, jnp.float32)

    @pl.loop(0, EPW, step=16)
    def _(j):
        d = dst_v[pl.ds(j, 16)]
        plsc.addupdate_scatter(deg_v, [d], ones16)
        pk_v[pl.ds(j, 16)] = (src_v[pl.ds(j, 16)] << SRC_SHIFT) | d

    pltpu.sync_copy(deg_v, deg_hbm.at[pl.ds(g * N, N)])
    pltpu.sync_copy(pk_v, pk_hbm.at[pl.ds(g * EPW, EPW)])


@functools.cache
def _sc_kernels():
    mesh = plsc.VectorSubcoreMesh(
        core_axis_name="c", subcore_axis_name="s",
        num_cores=NC, num_subcores=NS)
    params = pltpu.CompilerParams(needs_layout_passes=False,
                                  skip_device_barrier=True)
    common = [
        pltpu.VMEM((EPW,), jnp.int32),
        pltpu.VMEM((8, CH), jnp.int32),
        pltpu.VMEM((8, CH), jnp.int32),
        pltpu.VMEM((CH, D), jnp.float32),
        pltpu.VMEM((CH, D), jnp.float32),
    ]
    tail = [
        pltpu.SemaphoreType.DMA,
        pltpu.SemaphoreType.DMA,
        pltpu.VMEM_SHARED((N, D), jnp.float32),
    ]
    deg = pl.kernel(
        _sc_deg_body,
        compiler_params=params,
        out_type=(jax.ShapeDtypeStruct((NW * N,), jnp.float32),
                  jax.ShapeDtypeStruct((E,), jnp.int32)),
        mesh=mesh,
        scratch_types=[
            pltpu.VMEM((EPW,), jnp.int32),
            pltpu.VMEM((EPW,), jnp.int32),
            pltpu.VMEM((EPW,), jnp.int32),
            pltpu.VMEM((N,), jnp.float32),
        ],
    )
    agg = pl.kernel(
        functools.partial(_sc_agg_body, False),
        compiler_params=params,
        out_type=jax.ShapeDtypeStruct((NC, N, D), jnp.float32),
        mesh=mesh,
        scratch_types=common + tail,
    )
    return agg, deg


BM = 400
_GRID = N // BM


def _tc_layer_body(relu_out, acc_ref, deg_ref, x_ref, wl_ref, bl_ref, wr_ref,
                   *outs):
    deg = jnp.sum(deg_ref[...], axis=1, keepdims=True)
    invd = 1.0 / jnp.maximum(deg, 1.0)
    mean = (acc_ref[0] + acc_ref[1]) * invd
    h1 = (lax.dot_general(mean, wl_ref[...], (((1,), (1,)), ((), ())),
                          preferred_element_type=jnp.float32)
          + bl_ref[...]
          + lax.dot_general(x_ref[...], wr_ref[...], (((1,), (1,)), ((), ())),
                            preferred_element_type=jnp.float32))
    outs[0][...] = h1
    if relu_out:
        outs[1][...] = jnp.maximum(h1, 0.0)


def _make_tc(relu_out):
    n_out = 2 if relu_out else 1
    return pl.pallas_call(
        functools.partial(_tc_layer_body, relu_out),
        grid=(_GRID,),
        in_specs=[
            pl.BlockSpec((NC, BM, D), lambda i: (0, i, 0)),
            pl.BlockSpec((BM, NW), lambda i: (i, 0)),
            pl.BlockSpec((BM, D), lambda i: (i, 0)),
            pl.BlockSpec((D, D), lambda i: (0, 0)),
            pl.BlockSpec((1, D), lambda i: (0, 0)),
            pl.BlockSpec((D, D), lambda i: (0, 0)),
        ],
        out_specs=[pl.BlockSpec((BM, D), lambda i: (i, 0))] * n_out,
        out_shape=[jax.ShapeDtypeStruct((N, D), jnp.float32)] * n_out,
    )


_tc_layer_relu = _make_tc(True)
_tc_layer_last = _make_tc(False)


def kernel(x, edge_index, W_l0, b_l0, W_r0, W_l1, b_l1, W_r1):
    sc_agg, sc_deg = _sc_kernels()
    srcf = edge_index[0].astype(jnp.int32)
    dstf = edge_index[1].astype(jnp.int32)
    degp, pk = sc_deg(srcf, dstf)
    deg = degp.reshape(NW, N).T
    acc0 = sc_agg(x, pk)
    h1, h = _tc_layer_relu(acc0, deg, x, W_l0, b_l0.reshape(1, D), W_r0)
    acc1 = sc_agg(h, pk)
    (h2,) = _tc_layer_last(acc1, deg, h, W_l1, b_l1.reshape(1, D), W_r1)
    return (h1, h2)


# speedup vs baseline: 2.5029x; 1.4060x over previous
import functools

import jax
import jax.numpy as jnp
from jax import lax
from jax.experimental import pallas as pl
from jax.experimental.pallas import tpu as pltpu
from jax.experimental.pallas import tpu_sc as plsc

N = 10000
E = 320000
D = 128
DH = D // 2
NC = 2
NS = 16
NW = NC * NS
CH = 80
EPW = E // NW
ROWS_PW = EPW // CH
SRC_SHIFT = 14
NPT = N // NS


def _sc_agg_body(with_deg, x_hbm, pk_hbm, *rest):
    if with_deg:
        (acc_hbm, deg_hbm, pk_v, src_b, dst_b, rows_a, rows_b, rows_c, deg_v,
         sem_a, sem_b, sem_c, acc_sh) = rest
    else:
        (acc_hbm, pk_v, src_b, dst_b, rows_a, rows_b, rows_c,
         sem_a, sem_b, sem_c, acc_sh) = rest
        deg_hbm = deg_v = None
    c = lax.axis_index("c")
    s = lax.axis_index("s")
    g = c * NS + s

    z16 = jnp.zeros((16,), jnp.float32)
    ones16 = jnp.full((16,), 1.0, jnp.float32)

    if with_deg:
        @pl.loop(0, N, step=16)
        def _(i):
            deg_v[pl.ds(i, 16)] = z16

    @pl.loop(0, CH)
    def _(r):
        for k in range(D // 16):
            rows_a[r, pl.ds(k * 16, 16)] = z16

    for r in range(NPT // CH):
        pltpu.sync_copy(rows_a, acc_sh.at[pl.ds(s * NPT + r * CH, CH)])
    rem = NPT % CH
    pltpu.sync_copy(rows_a.at[pl.ds(0, rem)],
                    acc_sh.at[pl.ds(s * NPT + NPT - rem, rem)])
    plsc.subcore_barrier()

    pltpu.sync_copy(pk_hbm.at[pl.ds(g * EPW, EPW)], pk_v)

    def unpack(j, r):
        for k in range(CH // 16):
            p = pk_v[pl.ds(j * CH + k * 16, 16)]
            d = p & ((1 << SRC_SHIFT) - 1)
            src_b[r, pl.ds(k * 16, 16)] = p >> SRC_SHIFT
            dst_b[r, pl.ds(k * 16, 16)] = d
            if with_deg:
                plsc.addupdate_scatter(deg_v, [d], ones16)

    bufs = (rows_a, rows_b, rows_c)
    sems = (sem_a, sem_b, sem_c)

    def gather(r):
        pltpu.async_copy(x_hbm.at[src_b.at[r]], bufs[r], sems[r])

    def wait_gather(r):
        pltpu.make_async_copy(x_hbm.at[src_b.at[r]], bufs[r], sems[r]).wait()

    def scatter(r):
        pltpu.sync_copy(bufs[r], acc_sh.at[dst_b.at[r]], add=True)

    unpack(0, 0)
    gather(0)
    unpack(1, 1)
    gather(1)

    def step(j, r):
        unpack(j + 2, (r + 2) % 3)
        wait_gather(r)
        gather((r + 2) % 3)
        scatter(r)

    @pl.loop(0, (ROWS_PW - 2) // 3)
    def _(t):
        j = 3 * t
        step(j, 0)
        step(j + 1, 1)
        step(j + 2, 2)

    wait_gather(0)
    scatter(0)
    wait_gather(1)
    scatter(1)

    plsc.subcore_barrier()
    a8 = pl.multiple_of((s * NPT + 7) // 8 * 8, 8)
    b8 = ((s + 1) * NPT + 7) // 8 * 8
    pltpu.sync_copy(acc_sh.at[pl.ds(a8, 624)], acc_hbm.at[c, pl.ds(a8, 624)])

    @pl.when(b8 - a8 > 624)
    def _():
        t8 = pl.multiple_of(a8 + 624, 8)
        pltpu.sync_copy(acc_sh.at[pl.ds(t8, 8)], acc_hbm.at[c, pl.ds(t8, 8)])
    if with_deg:
        pltpu.sync_copy(deg_v, deg_hbm.at[pl.ds(g * N, N)])


def _sc_deg_body(src_hbm, dst_hbm, deg_hbm, pk_hbm, src_v, dst_v, pk_v,
                 deg_v):
    c = lax.axis_index("c")
    s = lax.axis_index("s")
    g = c * NS + s

    z16 = jnp.zeros((16,), jnp.float32)

    @pl.loop(0, N, step=16)
    def _(i):
        deg_v[pl.ds(i, 16)] = z16

    pltpu.sync_copy(src_hbm.at[pl.ds(g * EPW, EPW)], src_v)
    pltpu.sync_copy(dst_hbm.at[pl.ds(g * EPW, EPW)], dst_v)

    ones16 = jnp.full((16,), 1.0, jnp.float32)

    @pl.loop(0, EPW, step=16)
    def _(j):
        d = dst_v[pl.ds(j, 16)]
        plsc.addupdate_scatter(deg_v, [d], ones16)
        pk_v[pl.ds(j, 16)] = (src_v[pl.ds(j, 16)] << SRC_SHIFT) | d

    pltpu.sync_copy(deg_v, deg_hbm.at[pl.ds(g * N, N)])
    pltpu.sync_copy(pk_v, pk_hbm.at[pl.ds(g * EPW, EPW)])


@functools.cache
def _sc_kernels():
    mesh = plsc.VectorSubcoreMesh(
        core_axis_name="c", subcore_axis_name="s",
        num_cores=NC, num_subcores=NS)
    params = pltpu.CompilerParams(needs_layout_passes=False,
                                  skip_device_barrier=True)
    common = [
        pltpu.VMEM((EPW,), jnp.int32),
        pltpu.VMEM((8, CH), jnp.int32),
        pltpu.VMEM((8, CH), jnp.int32),
        pltpu.VMEM((CH, D), jnp.float32),
        pltpu.VMEM((CH, D), jnp.float32),
        pltpu.VMEM((CH, D), jnp.float32),
    ]
    tail = [
        pltpu.SemaphoreType.DMA,
        pltpu.SemaphoreType.DMA,
        pltpu.SemaphoreType.DMA,
        pltpu.VMEM_SHARED((N, D), jnp.float32),
    ]
    deg = pl.kernel(
        _sc_deg_body,
        compiler_params=params,
        out_type=(jax.ShapeDtypeStruct((NW * N,), jnp.float32),
                  jax.ShapeDtypeStruct((E,), jnp.int32)),
        mesh=mesh,
        scratch_types=[
            pltpu.VMEM((EPW,), jnp.int32),
            pltpu.VMEM((EPW,), jnp.int32),
            pltpu.VMEM((EPW,), jnp.int32),
            pltpu.VMEM((N,), jnp.float32),
        ],
    )
    agg = pl.kernel(
        functools.partial(_sc_agg_body, False),
        compiler_params=params,
        out_type=jax.ShapeDtypeStruct((NC, N, D), jnp.float32),
        mesh=mesh,
        scratch_types=common + tail,
    )
    return agg, deg


BM = 400
_GRID = N // BM


def _tc_layer_body(relu_out, acc_ref, deg_ref, x_ref, wl_ref, bl_ref, wr_ref,
                   *outs):
    deg = jnp.sum(deg_ref[...], axis=1, keepdims=True)
    invd = 1.0 / jnp.maximum(deg, 1.0)
    mean = (acc_ref[0] + acc_ref[1]) * invd
    h1 = (lax.dot_general(mean, wl_ref[...], (((1,), (1,)), ((), ())),
                          preferred_element_type=jnp.float32)
          + bl_ref[...]
          + lax.dot_general(x_ref[...], wr_ref[...], (((1,), (1,)), ((), ())),
                            preferred_element_type=jnp.float32))
    outs[0][...] = h1
    if relu_out:
        outs[1][...] = jnp.maximum(h1, 0.0)


def _make_tc(relu_out):
    n_out = 2 if relu_out else 1
    return pl.pallas_call(
        functools.partial(_tc_layer_body, relu_out),
        grid=(_GRID,),
        in_specs=[
            pl.BlockSpec((NC, BM, D), lambda i: (0, i, 0)),
            pl.BlockSpec((BM, NW), lambda i: (i, 0)),
            pl.BlockSpec((BM, D), lambda i: (i, 0)),
            pl.BlockSpec((D, D), lambda i: (0, 0)),
            pl.BlockSpec((1, D), lambda i: (0, 0)),
            pl.BlockSpec((D, D), lambda i: (0, 0)),
        ],
        out_specs=[pl.BlockSpec((BM, D), lambda i: (i, 0))] * n_out,
        out_shape=[jax.ShapeDtypeStruct((N, D), jnp.float32)] * n_out,
    )


_tc_layer_relu = _make_tc(True)
_tc_layer_last = _make_tc(False)


def kernel(x, edge_index, W_l0, b_l0, W_r0, W_l1, b_l1, W_r1):
    sc_agg, sc_deg = _sc_kernels()
    srcf = edge_index[0].astype(jnp.int32)
    dstf = edge_index[1].astype(jnp.int32)
    degp, pk = sc_deg(srcf, dstf)
    deg = degp.reshape(NW, N).T
    acc0 = sc_agg(x, pk)
    h1, h = _tc_layer_relu(acc0, deg, x, W_l0, b_l0.reshape(1, D), W_r0)
    acc1 = sc_agg(h, pk)
    (h2,) = _tc_layer_last(acc1, deg, h, W_l1, b_l1.reshape(1, D), W_r1)
    return (h1, h2)
